# baseline (speedup 1.0000x reference)
import functools

import jax
import jax.numpy as jnp
from jax.experimental import pallas as pl
from jax.experimental.pallas import tpu as pltpu


def _cox_body(t_col, t_row, p_col, p_row, ind_col, out_acc):
    i = pl.program_id(0)
    mask = t_row[...] >= t_col[...]
    e = jnp.exp(p_row[...])
    contrib = jnp.where(mask, e, jnp.zeros_like(e))
    s = jnp.sum(contrib, axis=1, keepdims=True)
    diffs = p_col[...] - jnp.log(s)
    ind = ind_col[...]
    num = jnp.sum(ind * diffs)
    den = jnp.sum(ind)
    vals = jnp.concatenate(
        [num.reshape(1, 1), den.reshape(1, 1)], axis=1)

    @pl.when(i == 0)
    def _init():
        out_acc[...] = jnp.zeros_like(out_acc)

    out_acc[...] += vals


@jax.jit
def kernel(pred, gt_indicator, gt_time):
    B = pred.shape[0]
    R = 256
    t_col = gt_time.reshape(B, 1)
    t_row = gt_time.reshape(1, B)
    p_col = pred.reshape(B, 1)
    p_row = pred.reshape(1, B)
    ind_col = gt_indicator.astype(jnp.float32).reshape(B, 1)

    acc = pl.pallas_call(
        _cox_body,
        grid=(B // R,),
        in_specs=[
            pl.BlockSpec((R, 1), lambda i: (i, 0)),
            pl.BlockSpec((1, B), lambda i: (0, 0)),
            pl.BlockSpec((R, 1), lambda i: (i, 0)),
            pl.BlockSpec((1, B), lambda i: (0, 0)),
            pl.BlockSpec((R, 1), lambda i: (i, 0)),
        ],
        out_specs=pl.BlockSpec((1, 2), lambda i: (0, 0)),
        out_shape=jax.ShapeDtypeStruct((1, 2), jnp.float32),
    )(t_col, t_row, p_col, p_row, ind_col)

    return -(acc[0, 0] / acc[0, 1])
